# baseline (speedup 1.0000x reference)
import functools

import jax
import jax.numpy as jnp
from jax import lax
from jax.experimental import pallas as pl
from jax.experimental.pallas import tpu as pltpu
from jax.experimental.pallas import tpu_sc as plsc

ALPHA = 0.2
B, L, V, D = 1024, 20, 100000, 64
BB = 8
BBL = BB * L
NSUB = 4
GB = BB * NSUB
GBL = GB * L
NEG = -9e15


@functools.cache
def _make_sc_gather():
    info = plsc.get_sparse_core_info()
    nc, ns = info.num_cores, info.num_subcores
    nw = nc * ns
    btot = B * L
    b_per_w = btot // nw
    mesh = plsc.VectorSubcoreMesh(core_axis_name="c", subcore_axis_name="s")

    @functools.partial(
        pl.kernel, mesh=mesh,
        out_type=jax.ShapeDtypeStruct((btot, 2 * D), jnp.float32),
        scratch_types=[
            pltpu.VMEM((b_per_w,), jnp.int32),
            pltpu.VMEM((b_per_w, 2 * D), jnp.float32),
            pltpu.SemaphoreType.DMA,
        ],
        compiler_params=pltpu.CompilerParams(use_tc_tiling_on_sc=False),
    )
    def gather_k(idx_hbm, table_hbm, out_hbm, idx_v, rows_v, sem):
        wid = lax.axis_index("s") * nc + lax.axis_index("c")
        base = wid * b_per_w
        pltpu.sync_copy(idx_hbm.at[pl.ds(base, b_per_w)], idx_v)
        pltpu.async_copy(table_hbm.at[idx_v], rows_v, sem).wait()
        pltpu.sync_copy(rows_v, out_hbm.at[pl.ds(base, b_per_w)])

    return gather_k


def _agg_sub(h, adj, a, tile_m, mask):
    hw = jnp.concatenate([h * a[k:k + 1, :] for k in range(4)], axis=0)
    e_all = lax.dot_general(hw, h, (((1,), (1,)), ((), ())),
                            preferred_element_type=jnp.float32)

    adj_t = lax.dot_general(adj, tile_m, (((1,), (0,)), ((), ())),
                            preferred_element_type=jnp.float32)

    base = jnp.where(adj_t == 1.0, e_all[0 * BBL:1 * BBL],
           jnp.where(adj_t == 2.0, e_all[1 * BBL:2 * BBL],
           jnp.where(adj_t == 3.0, e_all[2 * BBL:3 * BBL],
           jnp.where(adj_t == 4.0, e_all[3 * BBL:4 * BBL], NEG))))
    att = jnp.maximum(base, ALPHA * base) + mask

    m = jnp.max(att, axis=1, keepdims=True)
    p = jnp.exp(att - m)
    s = jnp.sum(p, axis=1, keepdims=True)

    acc = lax.dot_general(p, h, (((1,), (0,)), ((), ())),
                          preferred_element_type=jnp.float32)
    return acc * (1.0 / s)


def _agg_body(h_ref, adj_ref, a_ref, tile_ref, mask_ref, out_ref):
    a = a_ref[...]
    tile_m = tile_ref[...]
    mask = mask_ref[...]
    for sub in range(NSUB):
        r = slice(sub * BBL, (sub + 1) * BBL)
        h2 = h_ref[r, :]
        adjr = adj_ref[sub * BB:(sub + 1) * BB].reshape(BBL, L)
        par = adjr[:, 0:1] >> 3
        adj = (adjr & 7).astype(jnp.float32)
        h = jnp.where(par == 1, h2[:, D:], h2[:, :D])
        res = _agg_sub(h, adj, a, tile_m, mask)
        out_ref[sub * BB:(sub + 1) * BB] = res.reshape(BB, L, D)


_agg_call = pl.pallas_call(
    _agg_body,
    grid=(B // GB,),
    in_specs=[
        pl.BlockSpec((GBL, 2 * D), lambda i: (i, 0)),
        pl.BlockSpec((GB, L, L), lambda i: (i, 0, 0)),
        pl.BlockSpec((4, D), lambda i: (0, 0)),
        pl.BlockSpec((L, BBL), lambda i: (0, 0)),
        pl.BlockSpec((BBL, BBL), lambda i: (0, 0)),
    ],
    out_specs=pl.BlockSpec((GB, L, D), lambda i: (i, 0, 0)),
    out_shape=jax.ShapeDtypeStruct((B, L, D), jnp.float32),
)


def _consts():
    jl = jnp.arange(BBL)
    tile_m = (jl[None, :] % L == jnp.arange(L)[:, None]).astype(jnp.float32)
    same = (jl[:, None] // L) == (jl[None, :] // L)
    blk_mask = jnp.where(same, 0.0, -jnp.inf).astype(jnp.float32)
    return tile_m, blk_mask


def kernel(inputs, adj, mask_item, item, embedding, a_0, a_1, a_2, a_3):
    table = embedding.reshape(V // 2, 2 * D)
    idx2 = (inputs >> 1).reshape(-1)
    adj_p = adj | ((inputs & 1) << 3)[:, :, None]
    h2 = _make_sc_gather()(idx2, table)
    a_mat = jnp.concatenate([a_0, a_1, a_2, a_3], axis=1).T
    tile_m, blk_mask = _consts()
    return _agg_call(h2, adj_p, a_mat, tile_m, blk_mask)


# speedup vs baseline: 1.1041x; 1.1041x over previous
import functools

import jax
import jax.numpy as jnp
from jax import lax
from jax.experimental import pallas as pl
from jax.experimental.pallas import tpu as pltpu
from jax.experimental.pallas import tpu_sc as plsc

ALPHA = 0.2
B, L, V, D = 1024, 20, 100000, 64
BB = 8
BBL = BB * L
NSUB = 4
GB = BB * NSUB
GBL = GB * L
NEG = -9e15


@functools.cache
def _make_sc_gather():
    info = plsc.get_sparse_core_info()
    nc, ns = info.num_cores, info.num_subcores
    nw = nc * ns
    btot = B * L
    b_per_w = btot // nw
    mesh = plsc.VectorSubcoreMesh(core_axis_name="c", subcore_axis_name="s")

    @functools.partial(
        pl.kernel, mesh=mesh,
        out_type=jax.ShapeDtypeStruct((btot, 2 * D), jnp.float32),
        scratch_types=[
            pltpu.VMEM((b_per_w,), jnp.int32),
            pltpu.VMEM((b_per_w, D), jnp.float32),
            pltpu.SemaphoreType.DMA,
        ],
        compiler_params=pltpu.CompilerParams(use_tc_tiling_on_sc=False),
    )
    def gather_k(idx_hbm, table_hbm, out_hbm, idx_v, rows_v, sem):
        wid = lax.axis_index("s") * nc + lax.axis_index("c")
        base = wid * b_per_w
        pltpu.sync_copy(idx_hbm.at[pl.ds(base, b_per_w)], idx_v)
        pltpu.async_copy(table_hbm.at[idx_v], rows_v, sem).wait()
        pltpu.sync_copy(rows_v, out_hbm.at[pl.ds(base, b_per_w), pl.ds(0, D)])

    return gather_k


def _agg_sub(h, adj, a_rep, tile_m, mask):
    hw = jnp.concatenate([h, h, h, h], axis=0) * a_rep
    e_all = lax.dot_general(hw, h, (((1,), (1,)), ((), ())),
                            preferred_element_type=jnp.float32)

    adj_t = lax.dot_general(adj, tile_m, (((1,), (0,)), ((), ())),
                            preferred_element_type=jnp.float32)

    base = jnp.where(adj_t == 1.0, e_all[0 * BBL:1 * BBL],
           jnp.where(adj_t == 2.0, e_all[1 * BBL:2 * BBL],
           jnp.where(adj_t == 3.0, e_all[2 * BBL:3 * BBL],
           jnp.where(adj_t == 4.0, e_all[3 * BBL:4 * BBL], NEG))))
    att = jnp.maximum(base, ALPHA * base) + mask

    m = jnp.max(att, axis=1, keepdims=True)
    p = jnp.exp(att - m)
    s = jnp.sum(p, axis=1, keepdims=True)

    acc = lax.dot_general(p, h, (((1,), (0,)), ((), ())),
                          preferred_element_type=jnp.float32)
    return acc * (1.0 / s)


def _agg_body(h_ref, adj_ref, a_ref, tile_ref, mask_ref, out_ref):
    a_rep = a_ref[...]
    tile_m = tile_ref[...]
    mask = mask_ref[...]
    for sub in range(NSUB):
        r = slice(sub * BBL, (sub + 1) * BBL)
        h = h_ref[r, :D]
        adj = adj_ref[sub * BB:(sub + 1) * BB].astype(jnp.float32)
        adj = adj.reshape(BBL, L)
        res = _agg_sub(h, adj, a_rep, tile_m, mask)
        out_ref[sub * BB:(sub + 1) * BB] = res.reshape(BB, L, D)


_agg_call = pl.pallas_call(
    _agg_body,
    grid=(B // GB,),
    in_specs=[
        pl.BlockSpec((GBL, 2 * D), lambda i: (i, 0)),
        pl.BlockSpec((GB, L, L), lambda i: (i, 0, 0)),
        pl.BlockSpec((4 * BBL, D), lambda i: (0, 0)),
        pl.BlockSpec((L, BBL), lambda i: (0, 0)),
        pl.BlockSpec((BBL, BBL), lambda i: (0, 0)),
    ],
    out_specs=pl.BlockSpec((GB, L, D), lambda i: (i, 0, 0)),
    out_shape=jax.ShapeDtypeStruct((B, L, D), jnp.float32),
    compiler_params=pltpu.CompilerParams(dimension_semantics=("parallel",)),
)


def _consts():
    jl = jnp.arange(BBL)
    tile_m = (jl[None, :] % L == jnp.arange(L)[:, None]).astype(jnp.float32)
    same = (jl[:, None] // L) == (jl[None, :] // L)
    blk_mask = jnp.where(same, 0.0, -jnp.inf).astype(jnp.float32)
    return tile_m, blk_mask


def kernel(inputs, adj, mask_item, item, embedding, a_0, a_1, a_2, a_3):
    idx = inputs.reshape(-1)
    h2 = _make_sc_gather()(idx, embedding)
    a_mat = jnp.concatenate([a_0, a_1, a_2, a_3], axis=1).T
    a_rep = jnp.repeat(a_mat, BBL, axis=0)
    tile_m, blk_mask = _consts()
    return _agg_call(h2, adj, a_rep, tile_m, blk_mask)


# speedup vs baseline: 1.1092x; 1.0046x over previous
import functools

import jax
import jax.numpy as jnp
from jax import lax
from jax.experimental import pallas as pl
from jax.experimental.pallas import tpu as pltpu
from jax.experimental.pallas import tpu_sc as plsc

ALPHA = 0.2
B, L, V, D = 1024, 20, 100000, 64
BB = 8
BBL = BB * L
NSUB = 8
GB = BB * NSUB
GBL = GB * L
NEG = -9e15


@functools.cache
def _make_sc_gather():
    info = plsc.get_sparse_core_info()
    nc, ns = info.num_cores, info.num_subcores
    nw = nc * ns
    btot = B * L
    b_per_w = btot // nw
    mesh = plsc.VectorSubcoreMesh(core_axis_name="c", subcore_axis_name="s")

    @functools.partial(
        pl.kernel, mesh=mesh,
        out_type=jax.ShapeDtypeStruct((btot, 2 * D), jnp.float32),
        scratch_types=[
            pltpu.VMEM((b_per_w,), jnp.int32),
            pltpu.VMEM((b_per_w, D), jnp.float32),
            pltpu.SemaphoreType.DMA,
        ],
        compiler_params=pltpu.CompilerParams(use_tc_tiling_on_sc=False),
    )
    def gather_k(idx_hbm, table_hbm, out_hbm, idx_v, rows_v, sem):
        wid = lax.axis_index("s") * nc + lax.axis_index("c")
        base = wid * b_per_w
        pltpu.sync_copy(idx_hbm.at[pl.ds(base, b_per_w)], idx_v)
        pltpu.async_copy(table_hbm.at[idx_v], rows_v, sem).wait()
        pltpu.sync_copy(rows_v, out_hbm.at[pl.ds(base, b_per_w), pl.ds(0, D)])

    return gather_k


def _agg_sub(h, adj, a_rep, tile_m, mask):
    hw = jnp.concatenate([h, h, h, h], axis=0) * a_rep
    e_all = lax.dot_general(hw, h, (((1,), (1,)), ((), ())),
                            preferred_element_type=jnp.float32)

    adj_t = lax.dot_general(adj, tile_m, (((1,), (0,)), ((), ())),
                            preferred_element_type=jnp.float32)

    base = jnp.where(adj_t == 1.0, e_all[0 * BBL:1 * BBL],
           jnp.where(adj_t == 2.0, e_all[1 * BBL:2 * BBL],
           jnp.where(adj_t == 3.0, e_all[2 * BBL:3 * BBL],
           jnp.where(adj_t == 4.0, e_all[3 * BBL:4 * BBL], NEG))))
    att = jnp.maximum(base, ALPHA * base) + mask

    m = jnp.max(att, axis=1, keepdims=True)
    p = jnp.exp(att - m)
    s = jnp.sum(p, axis=1, keepdims=True)

    acc = lax.dot_general(p, h, (((1,), (0,)), ((), ())),
                          preferred_element_type=jnp.float32)
    return acc * (1.0 / s)


def _agg_body(h_ref, adj_ref, a_ref, tile_ref, mask_ref, out_ref):
    a_rep = a_ref[...]
    tile_m = tile_ref[...]
    mask = mask_ref[...]
    for sub in range(NSUB):
        r = slice(sub * BBL, (sub + 1) * BBL)
        h = h_ref[r, :D]
        adj = adj_ref[sub * BB:(sub + 1) * BB].astype(jnp.float32)
        adj = adj.reshape(BBL, L)
        res = _agg_sub(h, adj, a_rep, tile_m, mask)
        out_ref[sub * BB:(sub + 1) * BB] = res.reshape(BB, L, D)


_agg_call = pl.pallas_call(
    _agg_body,
    grid=(B // GB,),
    in_specs=[
        pl.BlockSpec((GBL, 2 * D), lambda i: (i, 0)),
        pl.BlockSpec((GB, L, L), lambda i: (i, 0, 0)),
        pl.BlockSpec((4 * BBL, D), lambda i: (0, 0)),
        pl.BlockSpec((L, BBL), lambda i: (0, 0)),
        pl.BlockSpec((BBL, BBL), lambda i: (0, 0)),
    ],
    out_specs=pl.BlockSpec((GB, L, D), lambda i: (i, 0, 0)),
    out_shape=jax.ShapeDtypeStruct((B, L, D), jnp.float32),
    compiler_params=pltpu.CompilerParams(dimension_semantics=("parallel",)),
)


def _consts():
    jl = jnp.arange(BBL)
    tile_m = (jl[None, :] % L == jnp.arange(L)[:, None]).astype(jnp.float32)
    same = (jl[:, None] // L) == (jl[None, :] // L)
    blk_mask = jnp.where(same, 0.0, -jnp.inf).astype(jnp.float32)
    return tile_m, blk_mask


def kernel(inputs, adj, mask_item, item, embedding, a_0, a_1, a_2, a_3):
    idx = inputs.reshape(-1)
    h2 = _make_sc_gather()(idx, embedding)
    a_mat = jnp.concatenate([a_0, a_1, a_2, a_3], axis=1).T
    a_rep = jnp.repeat(a_mat, BBL, axis=0)
    tile_m, blk_mask = _consts()
    return _agg_call(h2, adj, a_rep, tile_m, blk_mask)
